# initial kernel scaffold (unmeasured)
import jax
import jax.numpy as jnp
from jax import lax
from jax.experimental import pallas as pl
from jax.experimental.pallas import tpu as pltpu

N_DEV = 4
M_CH = 1024
N_COLS = 8192
NT = 4
N_TILE = N_COLS // NT


def kernel(x, w_mat, scale_x, scale_w):
    m, k = x.shape
    _, n = w_mat.shape

    def body(x_ref, w_ref, sx_ref, sw_ref, out_ref,
             comm_ref, send_sems, recv_sems):
        my = lax.axis_index("i")
        left = lax.rem(my + N_DEV - 1, N_DEV)
        right = lax.rem(my + 1, N_DEV)

        barrier_sem = pltpu.get_barrier_semaphore()
        for nbr in (left, right):
            pl.semaphore_signal(
                barrier_sem, inc=1,
                device_id=(nbr,), device_id_type=pl.DeviceIdType.MESH,
            )
        pl.semaphore_wait(barrier_sem, 2)

        def partial_tile(c, t):
            xs = x_ref[pl.ds(c * M_CH, M_CH), :]
            ws = w_ref[:, t * N_TILE:(t + 1) * N_TILE]
            acc = lax.dot_general(
                xs, ws, (((1,), (0,)), ((), ())),
                preferred_element_type=jnp.int32,
            )
            return acc.astype(jnp.float32)

        c0 = lax.rem(my + N_DEV - 1, N_DEV)
        for t in range(NT):
            comm_ref[0, :, t * N_TILE:(t + 1) * N_TILE] = partial_tile(c0, t)

        for s in range(N_DEV - 1):
            send_slot = s % 2
            recv_slot = (s + 1) % 2
            rdma = pltpu.make_async_remote_copy(
                src_ref=comm_ref.at[send_slot],
                dst_ref=comm_ref.at[recv_slot],
                send_sem=send_sems.at[send_slot],
                recv_sem=recv_sems.at[recv_slot],
                device_id=(right,),
                device_id_type=pl.DeviceIdType.MESH,
            )
            rdma.start()
            rdma.wait()

            r = lax.rem(my + 2 - s + N_DEV, N_DEV)
            if s < N_DEV - 2:
                for t in range(NT):
                    sl = slice(t * N_TILE, (t + 1) * N_TILE)
                    comm_ref[recv_slot, :, sl] = (
                        comm_ref[recv_slot, :, sl] + partial_tile(r, t)
                    )
            else:
                scale = sx_ref[0] * sw_ref[0]
                for t in range(NT):
                    sl = slice(t * N_TILE, (t + 1) * N_TILE)
                    acc = comm_ref[recv_slot, :, sl] + partial_tile(r, t)
                    y = acc * scale
                    z = jnp.clip(y, -60.0, 60.0)
                    out_ref[:, sl] = y / (1.0 + jnp.exp(-z))

    return pl.pallas_call(
        body,
        out_shape=jax.ShapeDtypeStruct((M_CH, n), jnp.float32),
        in_specs=[
            pl.BlockSpec(memory_space=pltpu.VMEM),
            pl.BlockSpec(memory_space=pltpu.VMEM),
            pl.BlockSpec(memory_space=pltpu.SMEM),
            pl.BlockSpec(memory_space=pltpu.SMEM),
        ],
        out_specs=pl.BlockSpec(memory_space=pltpu.VMEM),
        scratch_shapes=[
            pltpu.VMEM((2, M_CH, N_COLS), jnp.float32),
            pltpu.SemaphoreType.DMA((2,)),
            pltpu.SemaphoreType.DMA((2,)),
        ],
        compiler_params=pltpu.CompilerParams(collective_id=0),
    )(x, w_mat, scale_x, scale_w)


# baseline (device time: 669046 ns/iter reference)
import jax
import jax.numpy as jnp
from jax import lax
from jax.experimental import pallas as pl
from jax.experimental.pallas import tpu as pltpu

N_DEV = 4
M_CH = 1024
N_COLS = 8192
NT = 8
N_TILE = N_COLS // NT


def kernel(x, w_mat, scale_x, scale_w):
    m, k = x.shape
    _, n = w_mat.shape

    def body(x_ref, w_ref, sx_ref, sw_ref, out_ref,
             comm_ref, stage_ref, send_sems, recv_sems, out_sems):
        my = lax.axis_index("i")
        left = lax.rem(my + N_DEV - 1, N_DEV)
        right = lax.rem(my + 1, N_DEV)

        barrier_sem = pltpu.get_barrier_semaphore()
        for nbr in (left, right):
            pl.semaphore_signal(
                barrier_sem, inc=1,
                device_id=(nbr,), device_id_type=pl.DeviceIdType.MESH,
            )
        pl.semaphore_wait(barrier_sem, 2)

        def partial_tile(c, t):
            xs = x_ref[pl.ds(c * M_CH, M_CH), :]
            ws = w_ref[:, t * N_TILE:(t + 1) * N_TILE]
            acc = lax.dot_general(
                xs, ws, (((1,), (0,)), ((), ())),
                preferred_element_type=jnp.int32,
            )
            return acc.astype(jnp.float32)

        c0 = lax.rem(my + N_DEV - 1, N_DEV)
        for t in range(NT):
            sl = slice(t * N_TILE, (t + 1) * N_TILE)
            comm_ref[0, :, sl] = partial_tile(c0, t).astype(jnp.bfloat16)

        for s in range(N_DEV - 1):
            send_slot = s % 2
            recv_slot = (s + 1) % 2
            rdma = pltpu.make_async_remote_copy(
                src_ref=comm_ref.at[send_slot],
                dst_ref=comm_ref.at[recv_slot],
                send_sem=send_sems.at[send_slot],
                recv_sem=recv_sems.at[recv_slot],
                device_id=(right,),
                device_id_type=pl.DeviceIdType.MESH,
            )
            rdma.start()
            rdma.wait()

            r = lax.rem(my + 2 - s + N_DEV, N_DEV)
            if s < N_DEV - 2:
                for t in range(NT):
                    sl = slice(t * N_TILE, (t + 1) * N_TILE)
                    comm_ref[recv_slot, :, sl] = (
                        comm_ref[recv_slot, :, sl].astype(jnp.float32)
                        + partial_tile(r, t)
                    ).astype(jnp.bfloat16)
            else:
                scale = sx_ref[0] * sw_ref[0]
                copies = [None, None]
                for t in range(NT):
                    sl = slice(t * N_TILE, (t + 1) * N_TILE)
                    acc = (comm_ref[recv_slot, :, sl].astype(jnp.float32)
                           + partial_tile(r, t))
                    y = acc * scale
                    z = jnp.clip(y, -60.0, 60.0)
                    val = y / (1.0 + jnp.exp(-z))
                    slot = t % 2
                    if copies[slot] is not None:
                        copies[slot].wait()
                    stage_ref[slot] = val
                    cp = pltpu.make_async_copy(
                        stage_ref.at[slot],
                        out_ref.at[:, sl],
                        out_sems.at[slot],
                    )
                    cp.start()
                    copies[slot] = cp
                for cp in copies:
                    if cp is not None:
                        cp.wait()

    out = pl.pallas_call(
        body,
        out_shape=jax.ShapeDtypeStruct((M_CH, n), jnp.float32),
        in_specs=[
            pl.BlockSpec(memory_space=pltpu.VMEM),
            pl.BlockSpec(memory_space=pltpu.VMEM),
            pl.BlockSpec(memory_space=pltpu.SMEM),
            pl.BlockSpec(memory_space=pltpu.SMEM),
        ],
        out_specs=pl.BlockSpec(memory_space=pl.ANY),
        scratch_shapes=[
            pltpu.VMEM((2, M_CH, N_COLS), jnp.bfloat16),
            pltpu.VMEM((2, M_CH, N_TILE), jnp.float32),
            pltpu.SemaphoreType.DMA((2,)),
            pltpu.SemaphoreType.DMA((2,)),
            pltpu.SemaphoreType.DMA((2,)),
        ],
        compiler_params=pltpu.CompilerParams(
            collective_id=0,
            vmem_limit_bytes=64 * 1024 * 1024,
        ),
    )(x, w_mat, scale_x, scale_w)
    return out


# device time: 578815 ns/iter; 1.1559x vs baseline; 1.1559x over previous
import jax
import jax.numpy as jnp
from jax import lax
from jax.experimental import pallas as pl
from jax.experimental.pallas import tpu as pltpu

N_DEV = 4
M_CH = 1024
N_COLS = 8192
NT = 8
N_TILE = N_COLS // NT


def kernel(x, w_mat, scale_x, scale_w):
    m, k = x.shape
    _, n = w_mat.shape

    def body(x_ref, w_ref, sx_ref, sw_ref, out_ref,
             comm_ref, stage_ref, send_sems, recv_sems, out_sems):
        my = lax.axis_index("i")
        left = lax.rem(my + N_DEV - 1, N_DEV)
        right = lax.rem(my + 1, N_DEV)

        barrier_sem = pltpu.get_barrier_semaphore()
        for nbr in (left, right):
            pl.semaphore_signal(
                barrier_sem, inc=1,
                device_id=(nbr,), device_id_type=pl.DeviceIdType.MESH,
            )
        pl.semaphore_wait(barrier_sem, 2)

        def tile_sl(t):
            return pl.ds(t * N_TILE, N_TILE)

        def partial_tile(c, t):
            xs = x_ref[pl.ds(c * M_CH, M_CH), :]
            ws = w_ref[:, tile_sl(t)]
            acc = lax.dot_general(
                xs, ws, (((1,), (0,)), ((), ())),
                preferred_element_type=jnp.int32,
            )
            return acc.astype(jnp.float32)

        def hop_desc(h, t):
            src_slot = h % 2
            dst_slot = (h + 1) % 2
            return pltpu.make_async_remote_copy(
                src_ref=comm_ref.at[src_slot, :, tile_sl(t)],
                dst_ref=comm_ref.at[dst_slot, :, tile_sl(t)],
                send_sem=send_sems.at[src_slot, t],
                recv_sem=recv_sems.at[dst_slot, t],
                device_id=(right,),
                device_id_type=pl.DeviceIdType.MESH,
            )

        c0 = lax.rem(my + N_DEV - 1, N_DEV)
        for t in range(NT):
            comm_ref[0, :, tile_sl(t)] = partial_tile(c0, t).astype(jnp.bfloat16)
            hop_desc(0, t).start()

        for h in (1, 2):
            rs = h % 2
            rc = lax.rem(my + 2 - (h - 1) + N_DEV, N_DEV)
            for t in range(NT):
                p = partial_tile(rc, t)
                hop_desc(h - 1, t).wait_recv()
                comm_ref[rs, :, tile_sl(t)] = (
                    comm_ref[rs, :, tile_sl(t)].astype(jnp.float32) + p
                ).astype(jnp.bfloat16)
                if h == 2:
                    hop_desc(0, t).wait_send()
                hop_desc(h, t).start()

        scale = sx_ref[0] * sw_ref[0]
        copies = [None, None]
        for t in range(NT):
            p = partial_tile(my, t)
            hop_desc(2, t).wait_recv()
            acc = comm_ref[1, :, tile_sl(t)].astype(jnp.float32) + p
            y = acc * scale
            z = jnp.clip(y, -60.0, 60.0)
            val = y / (1.0 + jnp.exp(-z))
            slot = t % 2
            if copies[slot] is not None:
                copies[slot].wait()
            stage_ref[slot] = val
            cp = pltpu.make_async_copy(
                stage_ref.at[slot],
                out_ref.at[:, tile_sl(t)],
                out_sems.at[slot],
            )
            cp.start()
            copies[slot] = cp

        for t in range(NT):
            hop_desc(1, t).wait_send()
            hop_desc(2, t).wait_send()
        for cp in copies:
            if cp is not None:
                cp.wait()

    out = pl.pallas_call(
        body,
        out_shape=jax.ShapeDtypeStruct((M_CH, n), jnp.float32),
        in_specs=[
            pl.BlockSpec(memory_space=pltpu.VMEM),
            pl.BlockSpec(memory_space=pltpu.VMEM),
            pl.BlockSpec(memory_space=pltpu.SMEM),
            pl.BlockSpec(memory_space=pltpu.SMEM),
        ],
        out_specs=pl.BlockSpec(memory_space=pl.ANY),
        scratch_shapes=[
            pltpu.VMEM((2, M_CH, N_COLS), jnp.bfloat16),
            pltpu.VMEM((2, M_CH, N_TILE), jnp.float32),
            pltpu.SemaphoreType.DMA((2, NT)),
            pltpu.SemaphoreType.DMA((2, NT)),
            pltpu.SemaphoreType.DMA((2,)),
        ],
        compiler_params=pltpu.CompilerParams(
            collective_id=0,
            vmem_limit_bytes=64 * 1024 * 1024,
        ),
    )(x, w_mat, scale_x, scale_w)
    return out
